# Initial kernel scaffold; baseline (speedup 1.0000x reference)
import jax
import jax.numpy as jnp
from jax.experimental import pallas as pl


def kernel(x, edge_index, edge_weight, Wz, bz, Wr, br, Wh, bh, Wl, bl):
    raise NotImplementedError("write your pallas kernel here")


# speedup vs baseline: 1.3798x; 1.3798x over previous
import functools

import jax
import jax.numpy as jnp
from jax.experimental import pallas as pl

_BLOCK_ROWS = 1000


def _fused_cell_body(x_ref, az_ref, ah_ref, bz_ref, bh_ref, wl_ref, bl_ref,
                     o_ref):
    xb = x_ref[...]
    z = jax.nn.sigmoid(
        jnp.dot(xb, az_ref[...], preferred_element_type=jnp.float32)
        + bz_ref[...])
    ht = jnp.tanh(
        jnp.dot(xb, ah_ref[...], preferred_element_type=jnp.float32)
        + bh_ref[...])
    h = jnp.maximum((1.0 - z) * ht, 0.0)
    o_ref[...] = (
        jnp.dot(h, wl_ref[...], preferred_element_type=jnp.float32)
        + bl_ref[...])


def kernel(x, edge_index, edge_weight, Wz, bz, Wr, br, Wh, bh, Wl, bl):
    del edge_index, edge_weight, Wr, br
    n, f_in = x.shape
    f_out = Wz.shape[-1]

    az = (Wz[0, 0] + Wz[1, 0])[:f_in]
    ah = (Wh[0, 0] + Wh[1, 0])[:f_in]
    bz2 = bz.reshape(1, f_out)
    bh2 = bh.reshape(1, f_out)
    wl = Wl.reshape(f_out, 1)
    bl2 = bl.reshape(1, 1)

    grid = (pl.cdiv(n, _BLOCK_ROWS),)
    return pl.pallas_call(
        _fused_cell_body,
        grid=grid,
        in_specs=[
            pl.BlockSpec((_BLOCK_ROWS, f_in), lambda i: (i, 0)),
            pl.BlockSpec((f_in, f_out), lambda i: (0, 0)),
            pl.BlockSpec((f_in, f_out), lambda i: (0, 0)),
            pl.BlockSpec((1, f_out), lambda i: (0, 0)),
            pl.BlockSpec((1, f_out), lambda i: (0, 0)),
            pl.BlockSpec((f_out, 1), lambda i: (0, 0)),
            pl.BlockSpec((1, 1), lambda i: (0, 0)),
        ],
        out_specs=pl.BlockSpec((_BLOCK_ROWS, 1), lambda i: (i, 0)),
        out_shape=jax.ShapeDtypeStruct((n, 1), x.dtype),
    )(x, az, ah, bz2, bh2, wl, bl2)


# speedup vs baseline: 1.6429x; 1.1907x over previous
import functools

import jax
import jax.numpy as jnp
from jax.experimental import pallas as pl

_BLOCK_ROWS = 2000


def _fused_cell_body(x_ref, az_ref, ah_ref, bz_ref, bh_ref, wl_ref, bl_ref,
                     o_ref):
    xb = x_ref[...]
    z = jax.nn.sigmoid(
        jnp.dot(xb, az_ref[...], preferred_element_type=jnp.float32)
        + bz_ref[...])
    ht = jnp.tanh(
        jnp.dot(xb, ah_ref[...], preferred_element_type=jnp.float32)
        + bh_ref[...])
    h = jnp.maximum((1.0 - z) * ht, 0.0)
    o_ref[...] = (
        jnp.dot(h, wl_ref[...], preferred_element_type=jnp.float32)
        + bl_ref[...])


def kernel(x, edge_index, edge_weight, Wz, bz, Wr, br, Wh, bh, Wl, bl):
    del edge_index, edge_weight, Wr, br
    n, f_in = x.shape
    f_out = Wz.shape[-1]

    az = (Wz[0, 0] + Wz[1, 0])[:f_in]
    ah = (Wh[0, 0] + Wh[1, 0])[:f_in]
    bz2 = bz.reshape(1, f_out)
    bh2 = bh.reshape(1, f_out)
    wl = Wl.reshape(f_out, 1)
    bl2 = bl.reshape(1, 1)

    grid = (pl.cdiv(n, _BLOCK_ROWS),)
    return pl.pallas_call(
        _fused_cell_body,
        grid=grid,
        in_specs=[
            pl.BlockSpec((_BLOCK_ROWS, f_in), lambda i: (i, 0)),
            pl.BlockSpec((f_in, f_out), lambda i: (0, 0)),
            pl.BlockSpec((f_in, f_out), lambda i: (0, 0)),
            pl.BlockSpec((1, f_out), lambda i: (0, 0)),
            pl.BlockSpec((1, f_out), lambda i: (0, 0)),
            pl.BlockSpec((f_out, 1), lambda i: (0, 0)),
            pl.BlockSpec((1, 1), lambda i: (0, 0)),
        ],
        out_specs=pl.BlockSpec((_BLOCK_ROWS, 1), lambda i: (i, 0)),
        out_shape=jax.ShapeDtypeStruct((n, 1), x.dtype),
    )(x, az, ah, bz2, bh2, wl, bl2)


# speedup vs baseline: 1.6586x; 1.0096x over previous
import functools

import jax
import jax.numpy as jnp
from jax.experimental import pallas as pl

_BLOCK_ROWS = 5000


def _fused_cell_body(x_ref, az_ref, ah_ref, bz_ref, bh_ref, wl_ref, bl_ref,
                     o_ref):
    xb = x_ref[...]
    z = jax.nn.sigmoid(
        jnp.dot(xb, az_ref[...], preferred_element_type=jnp.float32)
        + bz_ref[...])
    ht = jnp.tanh(
        jnp.dot(xb, ah_ref[...], preferred_element_type=jnp.float32)
        + bh_ref[...])
    h = jnp.maximum((1.0 - z) * ht, 0.0)
    o_ref[...] = (
        jnp.dot(h, wl_ref[...], preferred_element_type=jnp.float32)
        + bl_ref[...])


def kernel(x, edge_index, edge_weight, Wz, bz, Wr, br, Wh, bh, Wl, bl):
    del edge_index, edge_weight, Wr, br
    n, f_in = x.shape
    f_out = Wz.shape[-1]

    az = (Wz[0, 0] + Wz[1, 0])[:f_in]
    ah = (Wh[0, 0] + Wh[1, 0])[:f_in]
    bz2 = bz.reshape(1, f_out)
    bh2 = bh.reshape(1, f_out)
    wl = Wl.reshape(f_out, 1)
    bl2 = bl.reshape(1, 1)

    grid = (pl.cdiv(n, _BLOCK_ROWS),)
    return pl.pallas_call(
        _fused_cell_body,
        grid=grid,
        in_specs=[
            pl.BlockSpec((_BLOCK_ROWS, f_in), lambda i: (i, 0)),
            pl.BlockSpec((f_in, f_out), lambda i: (0, 0)),
            pl.BlockSpec((f_in, f_out), lambda i: (0, 0)),
            pl.BlockSpec((1, f_out), lambda i: (0, 0)),
            pl.BlockSpec((1, f_out), lambda i: (0, 0)),
            pl.BlockSpec((f_out, 1), lambda i: (0, 0)),
            pl.BlockSpec((1, 1), lambda i: (0, 0)),
        ],
        out_specs=pl.BlockSpec((_BLOCK_ROWS, 1), lambda i: (i, 0)),
        out_shape=jax.ShapeDtypeStruct((n, 1), x.dtype),
    )(x, az, ah, bz2, bh2, wl, bl2)


# speedup vs baseline: 1.7055x; 1.0283x over previous
import jax
import jax.numpy as jnp
from jax.experimental import pallas as pl

_BLOCK_ROWS = 5000


def _fused_cell_body(x_ref, wz_ref, bz_ref, wh_ref, bh_ref, wl_ref, bl_ref,
                     o_ref):
    f_in = x_ref.shape[1]
    az = wz_ref[0, :f_in, :] + wz_ref[1, :f_in, :]
    ah = wh_ref[0, :f_in, :] + wh_ref[1, :f_in, :]
    aw = jnp.concatenate([az, ah], axis=1)
    f_out = az.shape[1]

    xb = x_ref[...]
    r = jnp.dot(xb, aw, preferred_element_type=jnp.float32)
    z = jax.nn.sigmoid(r[:, :f_out] + bz_ref[...])
    ht = jnp.tanh(r[:, f_out:] + bh_ref[...])
    h = jnp.maximum((1.0 - z) * ht, 0.0)
    o_ref[...] = (
        jnp.dot(h, wl_ref[...], preferred_element_type=jnp.float32)
        + bl_ref[...])


def kernel(x, edge_index, edge_weight, Wz, bz, Wr, br, Wh, bh, Wl, bl):
    del edge_index, edge_weight, Wr, br
    n, f_in = x.shape
    c_in = Wz.shape[-2]
    f_out = Wz.shape[-1]

    wz = Wz.reshape(2, c_in, f_out)
    wh = Wh.reshape(2, c_in, f_out)
    bz2 = bz.reshape(1, f_out)
    bh2 = bh.reshape(1, f_out)
    wl = Wl.reshape(f_out, 1)
    bl2 = bl.reshape(1, 1)

    grid = (pl.cdiv(n, _BLOCK_ROWS),)
    return pl.pallas_call(
        _fused_cell_body,
        grid=grid,
        in_specs=[
            pl.BlockSpec((_BLOCK_ROWS, f_in), lambda i: (i, 0)),
            pl.BlockSpec((2, c_in, f_out), lambda i: (0, 0, 0)),
            pl.BlockSpec((1, f_out), lambda i: (0, 0)),
            pl.BlockSpec((2, c_in, f_out), lambda i: (0, 0, 0)),
            pl.BlockSpec((1, f_out), lambda i: (0, 0)),
            pl.BlockSpec((f_out, 1), lambda i: (0, 0)),
            pl.BlockSpec((1, 1), lambda i: (0, 0)),
        ],
        out_specs=pl.BlockSpec((_BLOCK_ROWS, 1), lambda i: (i, 0)),
        out_shape=jax.ShapeDtypeStruct((n, 1), x.dtype),
    )(x, wz, bz2, wh, bh2, wl, bl2)
